# baseline (speedup 1.0000x reference)
import jax
import jax.numpy as jnp
from jax import lax
from jax.experimental import pallas as pl
from jax.experimental.pallas import tpu as pltpu
from jax.experimental.pallas import tpu_sc as plsc

B = 4096
D = 64
L = 16
VU = 100000
VC = 1000
VPAD = 100096
CHW = 6272
NCH = 16
LASTLO = VPAD - CHW
OUT1D = D * B
SHARE = OUT1D // 8
DUMP = OUT1D + 255
PADIDX = 1 << 30

_info = plsc.get_sparse_core_info()
_NC, _NS = _info.num_cores, _info.num_subcores
_NW = _NC * _NS


def _sc_scan_body(uids, iids, userT, itemT,
                  uoutA, uoutB, ioutA, ioutB,
                  idv, cidx, cb, vals, idxb, strip0,
                  spm_u, spm_i):
    s = lax.axis_index("s")
    core = lax.axis_index("c")
    is_user = s < _NS // 2
    c = (s & (_NS // 2 - 1)) * _NC + core
    zbase = (s & (_NS // 2 - 1)) * SHARE

    def phase_zero(ids_in, spm):
        pltpu.sync_copy(ids_in, idv.at[pl.ds(0, B)])
        zeros = jnp.zeros((L,), jnp.int32)
        pads = jnp.full((L,), PADIDX, jnp.int32)
        fzero = jnp.zeros((L,), jnp.float32)

        dump = jnp.full((L,), DUMP, jnp.int32)

        def pre(k, carry):
            cidx[pl.ds(k * L, L)] = zeros
            cb[pl.ds(k * L, L)] = pads
            vals[pl.ds(k * L, L)] = fzero
            idxb[pl.ds(k * L, L)] = dump
            return carry

        lax.fori_loop(0, B // L, pre, 0)

        def pre2(k, carry):
            vals[pl.ds(B + k * L, L)] = fzero
            idxb[pl.ds(B + k * L, L)] = dump
            return carry

        lax.fori_loop(0, 256 // L, pre2, 0)
        for j in range(SHARE // B):
            pltpu.sync_copy(vals.at[pl.ds(0, B)],
                            spm.at[pl.ds(zbase + j * B, B)])

    def phase_scan(tbl_in, spm):
        m_lo = c * CHW
        m_hi = m_lo + CHW
        lo_read = jnp.where(c == NCH - 1, LASTLO, m_lo)

        def comp(k, off):
            idx = idv[pl.ds(k * L, L)]
            m = (idx >= m_lo) & (idx < m_hi)
            lane = lax.broadcasted_iota(jnp.int32, (L,), 0) + k * L
            plsc.store_compressed(cidx.at[pl.ds(off, L)], idx - lo_read,
                                  mask=m)
            plsc.store_compressed(cb.at[pl.ds(off, L)], lane, mask=m)
            return off + plsc.all_reduce_population_count(m)[0]

        cnt = lax.fori_loop(0, B // L, comp, 0)
        nch = lax.div(cnt + (L - 1), L)
        nseg = lax.div(cnt + 255, 256)

        for g in range(8):
            strip = strip0
            pltpu.sync_copy(tbl_in.at[pl.ds(8 * g, 8), pl.ds(lo_read, CHW)],
                            strip)
            for rl in range(8):
                r = 8 * g + rl
                rsplat = jnp.full((L,), rl, jnp.int32)
                dump = jnp.full((L,), DUMP, jnp.int32)

                def gat(k, carry):
                    ks = pl.ds(k * L, L)
                    lc = cidx[ks]
                    vals[ks] = plsc.load_gather(strip, [rsplat, lc])
                    idxb[ks] = jnp.minimum(cb[ks] + r * B, dump)
                    return carry

                lax.fori_loop(0, nch, gat, 0)

                def sca(sg, carry):
                    ss = pl.ds(sg * 256, 256)
                    pltpu.sync_copy(vals.at[ss], spm.at[idxb.at[ss]])
                    return carry

                lax.fori_loop(0, nseg, sca, 0)

    def phase_export(spm, outa, outb):
        @pl.when(core == 0)
        def _expA():
            pltpu.sync_copy(spm.at[pl.ds(zbase, SHARE)],
                            outa.at[pl.ds(zbase, SHARE)])

        @pl.when(core == 1)
        def _expB():
            pltpu.sync_copy(spm.at[pl.ds(zbase, SHARE)],
                            outb.at[pl.ds(zbase, SHARE)])

    @pl.when(is_user)
    def _z1():
        phase_zero(uids, spm_u)

    @pl.when(jnp.logical_not(is_user))
    def _z2():
        phase_zero(iids, spm_i)

    plsc.subcore_barrier()

    @pl.when(is_user)
    def _s1():
        phase_scan(userT, spm_u)

    @pl.when(jnp.logical_not(is_user))
    def _s2():
        phase_scan(itemT, spm_i)

    plsc.subcore_barrier()

    @pl.when(is_user)
    def _e1():
        phase_export(spm_u, uoutA, uoutB)

    @pl.when(jnp.logical_not(is_user))
    def _e2():
        phase_export(spm_i, ioutA, ioutB)


@jax.jit
def _sc_scan(uids, iids, userT, itemT):
    f = pl.kernel(
        _sc_scan_body,
        out_type=(
            jax.ShapeDtypeStruct((OUT1D,), jnp.float32),
            jax.ShapeDtypeStruct((OUT1D,), jnp.float32),
            jax.ShapeDtypeStruct((OUT1D,), jnp.float32),
            jax.ShapeDtypeStruct((OUT1D,), jnp.float32),
        ),
        mesh=plsc.VectorSubcoreMesh(core_axis_name="c", subcore_axis_name="s"),
        scratch_types=[
            pltpu.VMEM((B + L,), jnp.int32),
            pltpu.VMEM((B + L,), jnp.int32),
            pltpu.VMEM((B + L,), jnp.int32),
            pltpu.VMEM((B + 256,), jnp.float32),
            pltpu.VMEM((B + 256,), jnp.int32),
            pltpu.VMEM((8, CHW), jnp.float32),
            pltpu.VMEM_SHARED((OUT1D + 256,), jnp.float32),
            pltpu.VMEM_SHARED((OUT1D + 256,), jnp.float32),
        ],
        compiler_params=pltpu.CompilerParams(use_tc_tiling_on_sc=True,
                                             needs_layout_passes=False),
    )
    return f(uids, iids, userT, itemT)


_BN = 512


def _tc_tags_cat_body(ttT_ref, tagsT_ref, catT_ref, cids_ref, out_ref):
    acc = jnp.dot(ttT_ref[...], tagsT_ref[...],
                  preferred_element_type=jnp.float32)
    iota = lax.broadcasted_iota(jnp.int32, (VC, _BN), 0)
    onehot = (iota == cids_ref[...][None, :]).astype(jnp.float32)
    acc = acc + jnp.dot(catT_ref[...], onehot,
                        preferred_element_type=jnp.float32)
    out_ref[...] = acc


@jax.jit
def _tc_tags_cat(ttT, tagsT, catT, cids):
    k = ttT.shape[1]
    return pl.pallas_call(
        _tc_tags_cat_body,
        grid=(B // _BN,),
        in_specs=[
            pl.BlockSpec((D, k), lambda i: (0, 0)),
            pl.BlockSpec((k, _BN), lambda i: (0, i)),
            pl.BlockSpec((D, VC), lambda i: (0, 0)),
            pl.BlockSpec((_BN,), lambda i: (i,)),
        ],
        out_specs=pl.BlockSpec((D, _BN), lambda i: (0, i)),
        out_shape=jax.ShapeDtypeStruct((D, B), jnp.float32),
        compiler_params=pltpu.CompilerParams(
            dimension_semantics=("arbitrary",),
        ),
    )(ttT, tagsT, catT, cids)


def _tc_add2_body(a_ref, b_ref, out_ref):
    out_ref[...] = a_ref[...] + b_ref[...]


@jax.jit
def _tc_add2(a, b):
    blk = OUT1D // 8
    return pl.pallas_call(
        _tc_add2_body,
        grid=(8,),
        in_specs=[
            pl.BlockSpec((blk,), lambda i: (i,)),
            pl.BlockSpec((blk,), lambda i: (i,)),
        ],
        out_specs=pl.BlockSpec((blk,), lambda i: (i,)),
        out_shape=jax.ShapeDtypeStruct((OUT1D,), jnp.float32),
        compiler_params=pltpu.CompilerParams(
            dimension_semantics=("arbitrary",),
        ),
    )(a, b)


def _tc_add3_body(a_ref, b_ref, c_ref, out_ref):
    out_ref[...] = a_ref[...] + b_ref[...] + c_ref[...]


@jax.jit
def _tc_add3(a, b, c):
    return pl.pallas_call(
        _tc_add3_body,
        grid=(B // _BN,),
        in_specs=[
            pl.BlockSpec((D, _BN), lambda i: (0, i)),
            pl.BlockSpec((D, _BN), lambda i: (0, i)),
            pl.BlockSpec((D, _BN), lambda i: (0, i)),
        ],
        out_specs=pl.BlockSpec((D, _BN), lambda i: (0, i)),
        out_shape=jax.ShapeDtypeStruct((D, B), jnp.float32),
        compiler_params=pltpu.CompilerParams(
            dimension_semantics=("arbitrary",),
        ),
    )(a, b, c)


def kernel(user_ids, item_ids, attr_category, attr_tags,
           user_table, item_table, category_table, tags_table):
    uids = user_ids.astype(jnp.int32)
    iids = item_ids.astype(jnp.int32)
    cids = attr_category.astype(jnp.int32)
    uA, uB, iA, iB = _sc_scan(uids, iids, user_table.T, item_table.T)
    tagcatT = _tc_tags_cat(tags_table.T, attr_tags.T, category_table.T, cids)
    user_embT = _tc_add2(uA, uB).reshape(D, B)
    item_totalT = _tc_add3(iA.reshape(D, B), iB.reshape(D, B), tagcatT)
    return (user_embT.T, item_totalT.T)


# speedup vs baseline: 1.3460x; 1.3460x over previous
import jax
import jax.numpy as jnp
from jax import lax
from jax.experimental import pallas as pl
from jax.experimental.pallas import tpu as pltpu
from jax.experimental.pallas import tpu_sc as plsc

B = 4096
D = 64
L = 16

_info = plsc.get_sparse_core_info()
_NC, _NS = _info.num_cores, _info.num_subcores
_NW = _NC * _NS
_BPW = B // _NW


def _extract(idv, rr):
    return idv[pl.ds(rr, L)][0]


def _sc_user_body(ids, table, out, idv, ob, sem):
    w = lax.axis_index("s") * _NC + lax.axis_index("c")
    base = w * _BPW
    sl = pl.ds(base, _BPW)
    pltpu.sync_copy(ids.at[sl], idv.at[pl.ds(0, _BPW)])

    def fire(rr, carry):
        pltpu.async_copy(table.at[_extract(idv, rr)], ob.at[rr], sem)
        return carry

    def drain(rr, carry):
        pltpu.make_async_copy(table.at[0], ob.at[rr], sem).wait()
        return carry

    lax.fori_loop(0, _BPW, fire, 0)
    lax.fori_loop(0, _BPW, drain, 0)
    pltpu.sync_copy(ob, out.at[sl])


def _sc_item_cat_body(iids, cids, table, cat_tbl, out,
                      idv, idv2, obi, obc, sem, semc):
    w = lax.axis_index("s") * _NC + lax.axis_index("c")
    base = w * _BPW
    sl = pl.ds(base, _BPW)
    pltpu.sync_copy(iids.at[sl], idv.at[pl.ds(0, _BPW)])
    pltpu.sync_copy(cids.at[sl], idv2.at[pl.ds(0, _BPW)])

    def fire(rr, carry):
        pltpu.async_copy(table.at[_extract(idv, rr)], obi.at[rr], sem)
        pltpu.async_copy(cat_tbl.at[_extract(idv2, rr)], obc.at[rr], semc)
        return carry

    def drain(rr, carry):
        pltpu.make_async_copy(table.at[0], obi.at[rr], sem).wait()
        pltpu.make_async_copy(cat_tbl.at[0], obc.at[rr], semc).wait()
        return carry

    def addloop(rr, carry):
        for c in range(D // L):
            cs = pl.ds(c * L, L)
            obi[rr, cs] = obi[rr, cs] + obc[rr, cs]
        return carry

    lax.fori_loop(0, _BPW, fire, 0)
    lax.fori_loop(0, _BPW, drain, 0)
    lax.fori_loop(0, _BPW, addloop, 0)
    pltpu.sync_copy(obi, out.at[sl])


_MESH = dict(core_axis_name="c", subcore_axis_name="s")


@jax.jit
def _sc_user(ids, table):
    f = pl.kernel(
        _sc_user_body,
        out_type=jax.ShapeDtypeStruct((B, D), jnp.float32),
        mesh=plsc.VectorSubcoreMesh(**_MESH),
        scratch_types=[
            pltpu.VMEM((_BPW + L,), jnp.int32),
            pltpu.VMEM((_BPW, D), jnp.float32),
            pltpu.SemaphoreType.DMA,
        ],
        compiler_params=pltpu.CompilerParams(use_tc_tiling_on_sc=True),
    )
    return f(ids, table)


@jax.jit
def _sc_item_cat(iids, cids, table, cat_tbl):
    f = pl.kernel(
        _sc_item_cat_body,
        out_type=jax.ShapeDtypeStruct((B, D), jnp.float32),
        mesh=plsc.VectorSubcoreMesh(**_MESH),
        scratch_types=[
            pltpu.VMEM((_BPW + L,), jnp.int32),
            pltpu.VMEM((_BPW + L,), jnp.int32),
            pltpu.VMEM((_BPW, D), jnp.float32),
            pltpu.VMEM((_BPW, D), jnp.float32),
            pltpu.SemaphoreType.DMA,
            pltpu.SemaphoreType.DMA,
        ],
        compiler_params=pltpu.CompilerParams(use_tc_tiling_on_sc=True),
    )
    return f(iids, cids, table, cat_tbl)


_BN = 512


def _tc_body(ttT_ref, tagsT_ref, ipc_ref, user_ref, out_ref, uout_ref):
    acc = jnp.dot(ttT_ref[...], tagsT_ref[...],
                  preferred_element_type=jnp.float32)
    out_ref[...] = acc + ipc_ref[...].T
    uout_ref[...] = user_ref[...].T


@jax.jit
def _tc_matmul_add(ttT, tagsT, ipc, user_rows):
    k = ttT.shape[1]
    return pl.pallas_call(
        _tc_body,
        grid=(B // _BN,),
        in_specs=[
            pl.BlockSpec((D, k), lambda i: (0, 0)),
            pl.BlockSpec((k, _BN), lambda i: (0, i)),
            pl.BlockSpec((_BN, D), lambda i: (i, 0)),
            pl.BlockSpec((_BN, D), lambda i: (i, 0)),
        ],
        out_specs=[
            pl.BlockSpec((D, _BN), lambda i: (0, i)),
            pl.BlockSpec((D, _BN), lambda i: (0, i)),
        ],
        out_shape=[
            jax.ShapeDtypeStruct((D, B), jnp.float32),
            jax.ShapeDtypeStruct((D, B), jnp.float32),
        ],
        compiler_params=pltpu.CompilerParams(
            dimension_semantics=("arbitrary",),
        ),
    )(ttT, tagsT, ipc, user_rows)


def kernel(user_ids, item_ids, attr_category, attr_tags,
           user_table, item_table, category_table, tags_table):
    uids = user_ids.astype(jnp.int32)
    iids = item_ids.astype(jnp.int32)
    cids = attr_category.astype(jnp.int32)
    user_rows = _sc_user(uids, user_table)
    ipc = _sc_item_cat(iids, cids, item_table, category_table)
    item_totalT, user_embT = _tc_matmul_add(
        tags_table.T, attr_tags.T, ipc, user_rows)
    return (user_embT.T, item_totalT.T)
